# baseline (device time: 36353 ns/iter reference)
import jax
import jax.numpy as jnp
from jax import lax
from jax.experimental import pallas as pl
from jax.experimental.pallas import tpu as pltpu

N_DEV = 4
B_PER = 2
SQ = 128
HG = 4
DH = 64
D_MODEL = 512
D_HEADS = HG * DH
HPP = 2
W_HALF = HPP * DH


def _unit_contrib(xb, w_u32, k_ref, v_ref, s):
    w_s = pltpu.bitcast(w_u32, jnp.bfloat16)
    wq = w_s[:, 0:W_HALF]
    wot = w_s[:, W_HALF:2 * W_HALF]
    q = jnp.dot(xb, wq, preferred_element_type=jnp.float32)
    ctx_rows = []
    for b in range(B_PER):
        ctx_heads = []
        for j in range(HPP):
            hh = s * HPP + j
            qbh = q[b * SQ:(b + 1) * SQ, j * DH:(j + 1) * DH]
            kb = k_ref[b, hh]
            sc = lax.dot_general(
                qbh, kb, (((1,), (1,)), ((), ())),
                preferred_element_type=jnp.float32,
            ) * 0.125
            sc = sc - jnp.max(sc, axis=1, keepdims=True)
            e = jnp.exp(sc)
            w = e / jnp.sum(e, axis=1, keepdims=True)
            vb = v_ref[b, hh]
            ctx_heads.append(jnp.dot(w, vb, preferred_element_type=jnp.float32))
        ctx_rows.append(jnp.concatenate(ctx_heads, axis=1))
    ctx = jnp.concatenate(ctx_rows, axis=0).astype(jnp.bfloat16)
    return lax.dot_general(
        ctx, wot, (((1,), (1,)), ((), ())),
        preferred_element_type=jnp.float32,
    )


def _body(x_ref, w_ref, k_hbm, v_hbm, out_ref, comm_ref, k_ref, v_ref,
          send_sems, recv_sems, dma_sems):
    my = lax.axis_index("i")
    right = lax.rem(my + 1, N_DEV)
    left = lax.rem(my + N_DEV - 1, N_DEV)
    b0 = my * B_PER

    kv_dmas = []
    for s16 in range(N_DEV * HG):
        off = (0, 3, 1, 2)[s16 // HG]
        g = lax.rem(my + off, N_DEV)
        hh = g * HG + (s16 % HG)
        pair = []
        for t, (hbm, scr) in enumerate(((k_hbm, k_ref), (v_hbm, v_ref))):
            d = pltpu.make_async_copy(
                hbm.at[pl.ds(b0, B_PER), :, hh, :],
                scr.at[:, s16],
                dma_sems.at[2 * s16 + t],
            )
            d.start()
            pair.append(d)
        kv_dmas.append(pair)

    barrier_sem = pltpu.get_barrier_semaphore()
    for nbr in (left, right):
        pl.semaphore_signal(
            barrier_sem, inc=1,
            device_id=(nbr,), device_id_type=pl.DeviceIdType.MESH,
        )
    pl.semaphore_wait(barrier_sem, 2)

    def wait_kv(s):
        for d in kv_dmas[2 * s] + kv_dmas[2 * s + 1]:
            d.wait()

    xb = x_ref[...].astype(jnp.bfloat16)

    def rdma(src_ref, dst_slot, send_idx, target):
        return pltpu.make_async_remote_copy(
            src_ref=src_ref,
            dst_ref=comm_ref.at[dst_slot],
            send_sem=send_sems.at[send_idx],
            recv_sem=recv_sems.at[dst_slot],
            device_id=(target,),
            device_id_type=pl.DeviceIdType.MESH,
        )

    cw0a = rdma(w_ref.at[0], 0, 0, right)
    cw0b = rdma(w_ref.at[1], 1, 1, right)
    ccw0a = rdma(w_ref.at[1], 3, 2, left)
    ccw0b = rdma(w_ref.at[0], 2, 3, left)
    cw0a.start()
    ccw0a.start()
    cw0b.start()
    ccw0b.start()

    wait_kv(0)
    acc = _unit_contrib(xb, w_ref[0], k_ref, v_ref, 0)

    cw0a.wait_recv()
    fwd_cw = rdma(comm_ref.at[0], 4, 4, right)
    fwd_cw.start()
    ccw0a.wait_recv()
    fwd_ccw = rdma(comm_ref.at[3], 5, 5, left)
    fwd_ccw.start()

    wait_kv(1)
    acc = acc + _unit_contrib(xb, w_ref[1], k_ref, v_ref, 1)
    wait_kv(2)
    acc = acc + _unit_contrib(xb, comm_ref[0], k_ref, v_ref, 2)
    wait_kv(5)
    acc = acc + _unit_contrib(xb, comm_ref[3], k_ref, v_ref, 5)

    cw0b.wait_recv()
    wait_kv(3)
    acc = acc + _unit_contrib(xb, comm_ref[1], k_ref, v_ref, 3)
    ccw0b.wait_recv()
    wait_kv(4)
    acc = acc + _unit_contrib(xb, comm_ref[2], k_ref, v_ref, 4)

    fwd_cw.wait_recv()
    wait_kv(6)
    acc = acc + _unit_contrib(xb, comm_ref[4], k_ref, v_ref, 6)
    fwd_ccw.wait_recv()
    wait_kv(7)
    acc = acc + _unit_contrib(xb, comm_ref[5], k_ref, v_ref, 7)

    for d in (cw0a, cw0b, ccw0a, ccw0b, fwd_cw, fwd_ccw):
        d.wait_send()

    out_ref[...] = acc


def kernel(x, Wq, K_ext, V_ext, Wo):
    xf = x.reshape(B_PER * SQ, D_MODEL)
    wq_h = Wq.astype(jnp.bfloat16).reshape(D_MODEL, HG // HPP, W_HALF).transpose(1, 0, 2)
    wot_h = Wo.T.astype(jnp.bfloat16).reshape(D_MODEL, HG // HPP, W_HALF).transpose(1, 0, 2)
    pack_bf = jnp.concatenate([wq_h, wot_h], axis=2)
    pack_u16 = lax.bitcast_convert_type(pack_bf, jnp.uint16)
    pack_u32 = lax.bitcast_convert_type(
        pack_u16.reshape(2, D_MODEL // 2, 2, D_HEADS).transpose(0, 1, 3, 2),
        jnp.uint32,
    )

    out = pl.pallas_call(
        _body,
        out_shape=jax.ShapeDtypeStruct((B_PER * SQ, D_MODEL), jnp.float32),
        in_specs=[
            pl.BlockSpec(memory_space=pltpu.VMEM),
            pl.BlockSpec(memory_space=pltpu.VMEM),
            pl.BlockSpec(memory_space=pl.ANY),
            pl.BlockSpec(memory_space=pl.ANY),
        ],
        out_specs=pl.BlockSpec(memory_space=pltpu.VMEM),
        scratch_shapes=[
            pltpu.VMEM((6, D_MODEL // 2, D_HEADS), jnp.uint32),
            pltpu.VMEM((B_PER, N_DEV * HG, SQ, DH), jnp.float32),
            pltpu.VMEM((B_PER, N_DEV * HG, SQ, DH), jnp.float32),
            pltpu.SemaphoreType.DMA((6,)),
            pltpu.SemaphoreType.DMA((6,)),
            pltpu.SemaphoreType.DMA((32,)),
        ],
        compiler_params=pltpu.CompilerParams(collective_id=0),
    )(xf, pack_u32, K_ext, V_ext)

    return out.reshape(B_PER, SQ, D_MODEL)


# device time: 22145 ns/iter; 1.6416x vs baseline; 1.6416x over previous
import jax
import jax.numpy as jnp
from jax import lax
from jax.experimental import pallas as pl
from jax.experimental.pallas import tpu as pltpu

N_DEV = 4
B_PER = 2
SQ = 128
HG = 4
DH = 64
D_MODEL = 512
D_HEADS = HG * DH
HPP = 2
W_HALF = HPP * DH

_GROUP_OFF = (0, 3, 1, 2)


def _unit_contrib(xb, w_u32, k_ref, v_ref, g, s):
    w_s = pltpu.bitcast(w_u32, jnp.bfloat16)
    wq = w_s[:, 0:W_HALF]
    wot = w_s[:, W_HALF:2 * W_HALF]
    q = jnp.dot(xb, wq, preferred_element_type=jnp.float32)
    ctx_rows = []
    for b in range(B_PER):
        ctx_heads = []
        for j in range(HPP):
            hh = g * HG + (s % 2) * HPP + j
            qbh = q[b * SQ:(b + 1) * SQ, j * DH:(j + 1) * DH]
            kt = k_ref[b, hh]
            sc = jnp.dot(qbh, kt, preferred_element_type=jnp.float32) * 0.125
            sc = sc - jnp.max(sc, axis=1, keepdims=True)
            e = jnp.exp(sc)
            w = e / jnp.sum(e, axis=1, keepdims=True)
            vt = v_ref[b, hh]
            ctx_heads.append(lax.dot_general(
                w, vt, (((1,), (1,)), ((), ())),
                preferred_element_type=jnp.float32,
            ))
        ctx_rows.append(jnp.concatenate(ctx_heads, axis=1))
    ctx = jnp.concatenate(ctx_rows, axis=0).astype(jnp.bfloat16)
    return lax.dot_general(
        ctx, wot, (((1,), (1,)), ((), ())),
        preferred_element_type=jnp.float32,
    )


def _body(x_ref, w_ref, k_ref, v_ref, out_ref, comm_ref, send_sems, recv_sems):
    my = lax.axis_index("i")
    right = lax.rem(my + 1, N_DEV)
    left = lax.rem(my + N_DEV - 1, N_DEV)
    groups = [lax.rem(my + off, N_DEV) for off in _GROUP_OFF]

    barrier_sem = pltpu.get_barrier_semaphore()
    for nbr in (left, right):
        pl.semaphore_signal(
            barrier_sem, inc=1,
            device_id=(nbr,), device_id_type=pl.DeviceIdType.MESH,
        )
    pl.semaphore_wait(barrier_sem, 2)

    xb = x_ref[...].astype(jnp.bfloat16)

    def rdma(src_ref, dst_slot, send_idx, target):
        return pltpu.make_async_remote_copy(
            src_ref=src_ref,
            dst_ref=comm_ref.at[dst_slot],
            send_sem=send_sems.at[send_idx],
            recv_sem=recv_sems.at[dst_slot],
            device_id=(target,),
            device_id_type=pl.DeviceIdType.MESH,
        )

    cw0a = rdma(w_ref.at[0], 0, 0, right)
    cw0b = rdma(w_ref.at[1], 1, 1, right)
    ccw0a = rdma(w_ref.at[1], 3, 2, left)
    ccw0b = rdma(w_ref.at[0], 2, 3, left)
    cw0a.start()
    ccw0a.start()
    cw0b.start()
    ccw0b.start()

    acc = _unit_contrib(xb, w_ref[0], k_ref, v_ref, groups[0], 0)

    cw0a.wait_recv()
    fwd_cw = rdma(comm_ref.at[0], 4, 4, right)
    fwd_cw.start()
    ccw0a.wait_recv()
    fwd_ccw = rdma(comm_ref.at[3], 5, 5, left)
    fwd_ccw.start()

    acc = acc + _unit_contrib(xb, w_ref[1], k_ref, v_ref, groups[0], 1)
    acc = acc + _unit_contrib(xb, comm_ref[0], k_ref, v_ref, groups[1], 2)
    acc = acc + _unit_contrib(xb, comm_ref[3], k_ref, v_ref, groups[2], 5)

    cw0b.wait_recv()
    acc = acc + _unit_contrib(xb, comm_ref[1], k_ref, v_ref, groups[1], 3)
    ccw0b.wait_recv()
    acc = acc + _unit_contrib(xb, comm_ref[2], k_ref, v_ref, groups[2], 4)

    fwd_cw.wait_recv()
    acc = acc + _unit_contrib(xb, comm_ref[4], k_ref, v_ref, groups[3], 6)
    fwd_ccw.wait_recv()
    acc = acc + _unit_contrib(xb, comm_ref[5], k_ref, v_ref, groups[3], 7)

    for d in (cw0a, cw0b, ccw0a, ccw0b, fwd_cw, fwd_ccw):
        d.wait_send()

    out_ref[...] = acc


def kernel(x, Wq, K_ext, V_ext, Wo):
    my = lax.axis_index("i")

    kb = lax.dynamic_slice_in_dim(K_ext, my * B_PER, B_PER, axis=0)
    vb = lax.dynamic_slice_in_dim(V_ext, my * B_PER, B_PER, axis=0)
    kt = jnp.transpose(kb, (0, 2, 3, 1))
    vt = jnp.transpose(vb, (0, 2, 3, 1))

    xf = x.reshape(B_PER * SQ, D_MODEL)
    wq_h = Wq.astype(jnp.bfloat16).reshape(D_MODEL, HG // HPP, W_HALF).transpose(1, 0, 2)
    wot_h = Wo.T.astype(jnp.bfloat16).reshape(D_MODEL, HG // HPP, W_HALF).transpose(1, 0, 2)
    pack_bf = jnp.concatenate([wq_h, wot_h], axis=2)
    pack_u16 = lax.bitcast_convert_type(pack_bf, jnp.uint16)
    pack_u32 = lax.bitcast_convert_type(
        pack_u16.reshape(2, D_MODEL // 2, 2, D_HEADS).transpose(0, 1, 3, 2),
        jnp.uint32,
    )

    out = pl.pallas_call(
        _body,
        out_shape=jax.ShapeDtypeStruct((B_PER * SQ, D_MODEL), jnp.float32),
        in_specs=[
            pl.BlockSpec(memory_space=pltpu.VMEM),
            pl.BlockSpec(memory_space=pltpu.VMEM),
            pl.BlockSpec(memory_space=pltpu.VMEM),
            pl.BlockSpec(memory_space=pltpu.VMEM),
        ],
        out_specs=pl.BlockSpec(memory_space=pltpu.VMEM),
        scratch_shapes=[
            pltpu.VMEM((6, D_MODEL // 2, D_HEADS), jnp.uint32),
            pltpu.SemaphoreType.DMA((6,)),
            pltpu.SemaphoreType.DMA((6,)),
        ],
        compiler_params=pltpu.CompilerParams(collective_id=0),
    )(xf, pack_u32, kt, vt)

    return out.reshape(B_PER, SQ, D_MODEL)
